# baseline (speedup 1.0000x reference)
import jax
import jax.numpy as jnp
from jax import lax
from jax.experimental import pallas as pl
from jax.experimental.pallas import tpu as pltpu
from jax.experimental.pallas import tpu_sc as plsc

D_MODEL = 128
BATCH = 4096
HIST = 50
N = BATCH * HIST

_info = plsc.get_sparse_core_info()
NC, NS = _info.num_cores, _info.num_subcores
NW = NC * NS
B_PER_W = N // NW
NBUF = 3
CHUNK = 320
NCHUNK = B_PER_W // CHUNK


def _gather_body(x_hbm, w_hbm, out_hbm, *refs):
    idxs = refs[0:NBUF]
    bufs = refs[NBUF:2 * NBUF]
    isems = refs[2 * NBUF:3 * NBUF]
    gsems = refs[3 * NBUF:4 * NBUF]
    ssems = refs[4 * NBUF:5 * NBUF]

    wid = lax.axis_index("s") * NC + lax.axis_index("c")
    base = wid * B_PER_W

    def icopy(i):
        return pltpu.async_copy(
            x_hbm.at[pl.ds(base + i * CHUNK, CHUNK)], idxs[i % NBUF],
            isems[i % NBUF])

    def gather(i):
        return pltpu.async_copy(w_hbm.at[idxs[i % NBUF]], bufs[i % NBUF],
                                gsems[i % NBUF])

    def store(i):
        return pltpu.async_copy(
            bufs[i % NBUF], out_hbm.at[pl.ds(base + i * CHUNK, CHUNK)],
            ssems[i % NBUF])

    ic = [None] * NCHUNK
    g = [None] * NCHUNK
    s = [None] * NCHUNK

    for j in range(NBUF):
        ic[j] = icopy(j)
    ic[0].wait()
    g[0] = gather(0)
    for i in range(NCHUNK):
        if i + 1 < NCHUNK:
            ic[i + 1].wait()
            if i + 1 - NBUF >= 0:
                s[i + 1 - NBUF].wait()
            g[i + 1] = gather(i + 1)
        g[i].wait()
        s[i] = store(i)
        if i + NBUF < NCHUNK:
            ic[i + NBUF] = icopy(i + NBUF)
    for i in range(max(0, NCHUNK - NBUF), NCHUNK):
        s[i].wait()


def kernel(x, W):
    xf = x.T.reshape(-1)
    mesh = plsc.VectorSubcoreMesh(core_axis_name="c", subcore_axis_name="s")
    out = pl.kernel(
        _gather_body,
        mesh=mesh,
        out_type=jax.ShapeDtypeStruct((N, D_MODEL), jnp.float32),
        scratch_types=(
            [pltpu.VMEM((CHUNK,), jnp.int32) for _ in range(NBUF)]
            + [pltpu.VMEM((CHUNK, D_MODEL), jnp.float32) for _ in range(NBUF)]
            + [pltpu.SemaphoreType.DMA for _ in range(3 * NBUF)]
        ),
    )(xf, W)
    return out.reshape(HIST, BATCH, D_MODEL).transpose(1, 0, 2)


# speedup vs baseline: 1.0080x; 1.0080x over previous
import jax
import jax.numpy as jnp
from jax import lax
from jax.experimental import pallas as pl
from jax.experimental.pallas import tpu as pltpu
from jax.experimental.pallas import tpu_sc as plsc

D_MODEL = 128
BATCH = 4096
HIST = 50
N = BATCH * HIST

_info = plsc.get_sparse_core_info()
NC, NS = _info.num_cores, _info.num_subcores
NW = NC * NS
B_PER_W = N // NW
NBUF = 3
CHUNK = 256
NCHUNK = B_PER_W // CHUNK


def _gather_body(x_hbm, w_hbm, out_hbm, idx_all, *refs):
    bufs = refs[0:NBUF]
    gsems = refs[NBUF:2 * NBUF]
    ssems = refs[2 * NBUF:3 * NBUF]

    wid = lax.axis_index("s") * NC + lax.axis_index("c")
    base = wid * B_PER_W

    pltpu.sync_copy(x_hbm.at[pl.ds(base, B_PER_W)], idx_all)

    def gather(i):
        return pltpu.async_copy(
            w_hbm.at[idx_all.at[pl.ds(i * CHUNK, CHUNK)]], bufs[i % NBUF],
            gsems[i % NBUF])

    def store(i):
        return pltpu.async_copy(
            bufs[i % NBUF], out_hbm.at[pl.ds(base + i * CHUNK, CHUNK)],
            ssems[i % NBUF])

    g = [None] * NCHUNK
    s = [None] * NCHUNK

    g[0] = gather(0)
    for i in range(NCHUNK):
        if i + 1 < NCHUNK:
            if i + 1 - NBUF >= 0:
                s[i + 1 - NBUF].wait()
            g[i + 1] = gather(i + 1)
        g[i].wait()
        s[i] = store(i)
    for i in range(max(0, NCHUNK - NBUF), NCHUNK):
        s[i].wait()


def kernel(x, W):
    xf = x.T.reshape(-1)
    mesh = plsc.VectorSubcoreMesh(core_axis_name="c", subcore_axis_name="s")
    out = pl.kernel(
        _gather_body,
        mesh=mesh,
        out_type=jax.ShapeDtypeStruct((N, D_MODEL), jnp.float32),
        scratch_types=(
            [pltpu.VMEM((B_PER_W,), jnp.int32)]
            + [pltpu.VMEM((CHUNK, D_MODEL), jnp.float32) for _ in range(NBUF)]
            + [pltpu.SemaphoreType.DMA for _ in range(2 * NBUF)]
        ),
    )(xf, W)
    return out.reshape(HIST, BATCH, D_MODEL).transpose(1, 0, 2)


# speedup vs baseline: 1.0222x; 1.0141x over previous
import jax
import jax.numpy as jnp
from jax import lax
from jax.experimental import pallas as pl
from jax.experimental.pallas import tpu as pltpu
from jax.experimental.pallas import tpu_sc as plsc

D_MODEL = 128
BATCH = 4096
HIST = 50
N = BATCH * HIST

_info = plsc.get_sparse_core_info()
NC, NS = _info.num_cores, _info.num_subcores
NW = NC * NS
B_PER_W = N // NW
CHUNK = 400
NCHUNK = B_PER_W // CHUNK


def _gather_body(x_hbm, w_hbm, out_hbm, idx0, idx1, buf0, buf1,
                 isem0, isem1, gsem0, gsem1, ssem0, ssem1):
    wid = lax.axis_index("s") * NC + lax.axis_index("c")
    base = wid * B_PER_W

    idxs = (idx0, idx1)
    bufs = (buf0, buf1)
    isems = (isem0, isem1)
    gsems = (gsem0, gsem1)
    ssems = (ssem0, ssem1)

    def icopy(i):
        return pltpu.async_copy(
            x_hbm.at[pl.ds(base + i * CHUNK, CHUNK)], idxs[i % 2],
            isems[i % 2])

    def gather(i):
        return pltpu.async_copy(w_hbm.at[idxs[i % 2]], bufs[i % 2],
                                gsems[i % 2])

    def store(i):
        return pltpu.async_copy(
            bufs[i % 2], out_hbm.at[pl.ds(base + i * CHUNK, CHUNK)],
            ssems[i % 2])

    ic = [None] * NCHUNK
    g = [None] * NCHUNK
    s = [None] * NCHUNK

    ic[0] = icopy(0)
    ic[1] = icopy(1)
    ic[0].wait()
    g[0] = gather(0)
    for i in range(NCHUNK):
        if i + 1 < NCHUNK:
            ic[i + 1].wait()
            if i >= 1:
                s[i - 1].wait()
            g[i + 1] = gather(i + 1)
        g[i].wait()
        s[i] = store(i)
        if i + 2 < NCHUNK:
            ic[i + 2] = icopy(i + 2)
    s[NCHUNK - 2].wait()
    s[NCHUNK - 1].wait()


def kernel(x, W):
    xf = x.T.reshape(-1)
    mesh = plsc.VectorSubcoreMesh(core_axis_name="c", subcore_axis_name="s")
    out = pl.kernel(
        _gather_body,
        mesh=mesh,
        out_type=jax.ShapeDtypeStruct((N, D_MODEL), jnp.float32),
        scratch_types=[
            pltpu.VMEM((CHUNK,), jnp.int32),
            pltpu.VMEM((CHUNK,), jnp.int32),
            pltpu.VMEM((CHUNK, D_MODEL), jnp.float32),
            pltpu.VMEM((CHUNK, D_MODEL), jnp.float32),
            pltpu.SemaphoreType.DMA,
            pltpu.SemaphoreType.DMA,
            pltpu.SemaphoreType.DMA,
            pltpu.SemaphoreType.DMA,
            pltpu.SemaphoreType.DMA,
            pltpu.SemaphoreType.DMA,
        ],
    )(xf, W)
    return out.reshape(HIST, BATCH, D_MODEL).transpose(1, 0, 2)
